# Initial kernel scaffold; baseline (speedup 1.0000x reference)
import jax
import jax.numpy as jnp
from jax.experimental import pallas as pl


def kernel(x, edge_index, edge_type, edge_attr, batch, W1, q1, k1, We1, e1, b1, W2, q2, k2, We2, e2, b2, W3, q3, k3, We3, e3, b3, lin_w, lin_b):
    raise NotImplementedError("write your pallas kernel here")


# speedup vs baseline: 1.0224x; 1.0224x over previous
import functools

import jax
import jax.numpy as jnp
from jax.experimental import pallas as pl
from jax.experimental.pallas import tpu as pltpu

N = 10000
E = 160000
HEADS = 8
OUT = 128
R = 3
HID = HEADS * OUT
G = 16
ACT = 16

NPAD = 10240


def _proj_body(x_ref, w_ref, q_ref, k_ref, xr_ref, qn_ref, kn_ref, acc_ref):
    kt = pl.program_id(2)
    nk = pl.num_programs(2)

    @pl.when(kt == 0)
    def _():
        acc_ref[...] = jnp.zeros_like(acc_ref)

    acc_ref[...] += jnp.dot(x_ref[...], w_ref[0], preferred_element_type=jnp.float32)

    @pl.when(kt == nk - 1)
    def _():
        xr = acc_ref[...]
        xr_ref[0] = xr
        qn_ref[0] = jnp.dot(xr, q_ref[...], preferred_element_type=jnp.float32)
        kn_ref[0] = jnp.dot(xr, k_ref[...], preferred_element_type=jnp.float32)


def _project(x, W, q, k):
    K = x.shape[1]
    BN = 512
    BK = 128
    grid = (R, NPAD // BN, K // BK)
    return pl.pallas_call(
        _proj_body,
        grid=grid,
        in_specs=[
            pl.BlockSpec((BN, BK), lambda r, n, kt: (n, kt)),
            pl.BlockSpec((1, BK, HID), lambda r, n, kt: (r, kt, 0)),
            pl.BlockSpec((HID, HEADS), lambda r, n, kt: (0, 0)),
            pl.BlockSpec((HID, HEADS), lambda r, n, kt: (0, 0)),
        ],
        out_specs=[
            pl.BlockSpec((1, BN, HID), lambda r, n, kt: (r, n, 0)),
            pl.BlockSpec((1, BN, HEADS), lambda r, n, kt: (r, n, 0)),
            pl.BlockSpec((1, BN, HEADS), lambda r, n, kt: (r, n, 0)),
        ],
        out_shape=[
            jax.ShapeDtypeStruct((R, NPAD, HID), jnp.float32),
            jax.ShapeDtypeStruct((R, NPAD, HEADS), jnp.float32),
            jax.ShapeDtypeStruct((R, NPAD, HEADS), jnp.float32),
        ],
        scratch_shapes=[pltpu.VMEM((BN, HID), jnp.float32)],
    )(x, W, q, k)


def _conv(xp, edge_index, edge_type, edge_attr, W, q, k, We, e, b):
    src = edge_index[0]
    dst = edge_index[1]
    x_rel, q_nodes, k_nodes = _project(xp, W, q, k)
    qi = q_nodes[edge_type, dst]
    kj = k_nodes[edge_type, src]
    ee = jnp.matmul(jnp.matmul(edge_attr, We), e)
    alpha = jax.nn.leaky_relu(qi + kj + ee, negative_slope=0.2)
    ex = jnp.exp(alpha)
    denom = jax.ops.segment_sum(ex, dst, num_segments=N)
    att = ex / (denom[dst] + 1e-16)
    xj = x_rel[edge_type, src].reshape(-1, HEADS, OUT)
    msg = att[:, :, None] * xj
    out = jax.ops.segment_sum(msg, dst, num_segments=N).reshape(N, HID) + b
    return jnp.pad(jax.nn.relu(out), ((0, NPAD - N), (0, 0)))


def kernel(x, edge_index, edge_type, edge_attr, batch, W1, q1, k1, We1, e1, b1,
           W2, q2, k2, We2, e2, b2, W3, q3, k3, We3, e3, b3, lin_w, lin_b):
    xp = jnp.pad(x, ((0, NPAD - N), (0, 0)))
    h = _conv(xp, edge_index, edge_type, edge_attr, W1, q1, k1, We1, e1, b1)
    h = _conv(h, edge_index, edge_type, edge_attr, W2, q2, k2, We2, e2, b2)
    h = _conv(h, edge_index, edge_type, edge_attr, W3, q3, k3, We3, e3, b3)
    h = h[:N]
    t = jnp.tanh(jnp.matmul(h, lin_w) + lin_b)
    sums = jax.ops.segment_sum(t, batch, num_segments=G)
    cnt = jax.ops.segment_sum(jnp.ones((N, 1), t.dtype), batch, num_segments=G)
    return sums / jnp.maximum(cnt, 1.0)
